# baseline (speedup 1.0000x reference)
import jax
import jax.numpy as jnp
from jax.experimental import pallas as pl
from jax.experimental.pallas import tpu as pltpu


def _mlp_body(h_ref, gid_ref, mask_ref, w1_ref, b1_ref, w2_ref, b2_ref, o_ref):
    T = h_ref.shape[0]
    GH = w1_ref.shape[1]
    H = 64
    G = GH // H
    P = o_ref.shape[1]

    x = h_ref[...]
    h1 = jnp.dot(x, w1_ref[...], preferred_element_type=jnp.float32) + b1_ref[...]
    h1 = jnp.maximum(h1, 0.0)

    gid = gid_ref[...]
    acc = jnp.zeros((T, P), jnp.float32)
    for g in range(G):
        lg = (
            jnp.dot(h1[:, g * H : (g + 1) * H], w2_ref[g * H : (g + 1) * H, :],
                    preferred_element_type=jnp.float32)
            + b2_ref[g : g + 1, :]
        )
        acc = acc + jnp.where(gid == g, lg, 0.0)

    logits = jnp.where(mask_ref[...] > 0, acc, -1e9)
    m = jnp.max(logits, axis=1, keepdims=True)
    e = jnp.exp(logits - m)
    o_ref[...] = e / jnp.sum(e, axis=1, keepdims=True)


def kernel(h_int, group_ids, feasible_mask, W1, b1, W2, b2):
    B, D = h_int.shape
    G, _, H = W1.shape
    P = W2.shape[2]
    T = 512

    W1r = W1.transpose(1, 0, 2).reshape(D, G * H)
    b1r = b1.reshape(1, G * H)
    W2r = W2.reshape(G * H, P)
    gid2 = group_ids.reshape(B, 1)
    maskf = feasible_mask.astype(jnp.float32)

    out = pl.pallas_call(
        _mlp_body,
        grid=(B // T,),
        in_specs=[
            pl.BlockSpec((T, D), lambda i: (i, 0)),
            pl.BlockSpec((T, 1), lambda i: (i, 0)),
            pl.BlockSpec((T, P), lambda i: (i, 0)),
            pl.BlockSpec((D, G * H), lambda i: (0, 0)),
            pl.BlockSpec((1, G * H), lambda i: (0, 0)),
            pl.BlockSpec((G * H, P), lambda i: (0, 0)),
            pl.BlockSpec((G, P), lambda i: (0, 0)),
        ],
        out_specs=pl.BlockSpec((T, P), lambda i: (i, 0)),
        out_shape=jax.ShapeDtypeStruct((B, P), jnp.float32),
    )(h_int, gid2, maskf, W1r, b1r, W2r, b2)
    return out
